# Initial kernel scaffold; baseline (speedup 1.0000x reference)
import jax
import jax.numpy as jnp
from jax.experimental import pallas as pl


def kernel(edge_values, emb, W_mean, b_mean, W_std, b_std, Wd1, bd1, Wd2, bd2, noise, edge_index, users, items, neg_items):
    raise NotImplementedError("write your pallas kernel here")


# speedup vs baseline: 4.4631x; 4.4631x over previous
import dataclasses
import functools

import jax
import jax.numpy as jnp
from jax import lax
from jax.experimental import pallas as pl
from jax.experimental.pallas import tpu as pltpu
from jax.experimental.pallas import tpu_sc as plsc

N_NODES = 10000
N_USER = 5000
N_EDGES = 320000
D = 128
B = 4096

NC = 2
NS = 16
LANES = 16
CHUNK = 128
CHUNKS_PER_CORE = N_EDGES // CHUNK // NC
ROWS_PER_SUBCORE = 624


def _mesh():
    return plsc.VectorSubcoreMesh(core_axis_name="c", subcore_axis_name="s")


def _sc_compiler_params():
    cp = pltpu.CompilerParams()
    if "needs_layout_passes" in pltpu.CompilerParams.__dataclass_fields__:
        cp = dataclasses.replace(cp, needs_layout_passes=False)
    return cp


def _sc_spmm(row, col, ev, table):
    @functools.partial(
        pl.kernel,
        out_type=jax.ShapeDtypeStruct((NC, N_NODES, D), jnp.float32),
        mesh=_mesh(),
        scratch_types=[
            pltpu.VMEM((CHUNK,), jnp.int32),
            pltpu.VMEM((CHUNK,), jnp.int32),
            pltpu.VMEM((CHUNK,), jnp.float32),
            pltpu.VMEM((CHUNK, D), jnp.float32),
            pltpu.VMEM_SHARED((N_NODES, D), jnp.float32),
            pltpu.SemaphoreType.DMA,
        ],
        compiler_params=_sc_compiler_params(),
    )
    def spmm(row_h, col_h, ev_h, tab_h, out_h, colv, rowv, evv, rows, acc, sem):
        c = lax.axis_index("c")
        s = lax.axis_index("s")

        zero16 = jnp.zeros((LANES,), jnp.float32)

        @pl.loop(0, CHUNK)
        def _(k):
            for d in range(D // LANES):
                rows[k, pl.ds(d * LANES, LANES)] = zero16

        base_row = s * ROWS_PER_SUBCORE
        for j in range(4):
            pltpu.sync_copy(rows.at[pl.ds(0, CHUNK)],
                            acc.at[pl.ds(base_row + j * CHUNK, CHUNK)])
        pltpu.sync_copy(rows.at[pl.ds(0, 112)],
                        acc.at[pl.ds(base_row + 4 * CHUNK, 112)])

        @pl.when(s == NS - 1)
        def _():
            pltpu.sync_copy(rows.at[pl.ds(0, 16)],
                            acc.at[pl.ds(N_NODES - 16, 16)])

        plsc.subcore_barrier()

        n_i = (CHUNKS_PER_CORE + NS - 1 - s) // NS

        @pl.loop(0, n_i)
        def _(i):
            chunk = c * CHUNKS_PER_CORE + s + i * NS
            off = chunk * CHUNK
            pltpu.sync_copy(col_h.at[pl.ds(off, CHUNK)], colv)
            pltpu.sync_copy(row_h.at[pl.ds(off, CHUNK)], rowv)
            pltpu.sync_copy(ev_h.at[pl.ds(off, CHUNK)], evv)
            pltpu.async_copy(tab_h.at[colv], rows, sem).wait()

            @pl.loop(0, CHUNK)
            def _(k):
                evk = plsc.load_gather(evv, [jnp.full((LANES,), k, jnp.int32)])
                for d in range(D // LANES):
                    sl = pl.ds(d * LANES, LANES)
                    rows[k, sl] = rows[k, sl] * evk

            pltpu.sync_copy(rows, acc.at[rowv], add=True)

        plsc.subcore_barrier()
        pltpu.sync_copy(acc.at[pl.ds(base_row, ROWS_PER_SUBCORE)],
                        out_h.at[c, pl.ds(base_row, ROWS_PER_SUBCORE)])

        @pl.when(s == NS - 1)
        def _():
            pltpu.sync_copy(acc.at[pl.ds(N_NODES - 16, 16)],
                            out_h.at[c, pl.ds(N_NODES - 16, 16)])

    return spmm(row, col, ev, table)


def _tc_add(p):
    def body(p_ref, o_ref):
        o_ref[...] = p_ref[0] + p_ref[1]

    return pl.pallas_call(
        body,
        out_shape=jax.ShapeDtypeStruct((N_NODES, D), jnp.float32),
    )(p)


def _softplus(t):
    return jnp.maximum(t, 0.0) + jnp.log(1.0 + jnp.exp(-jnp.abs(t)))


ROW_BLK = 1000


def _tc_encode(q, noise, W_mean, b_mean, W_std, b_std):
    nblk = N_NODES // ROW_BLK

    def body(q_ref, n_ref, wm_ref, bm_ref, ws_ref, bs_ref, x_ref, kl_ref, acc_ref):
        i = pl.program_id(0)
        h2 = q_ref[0] + q_ref[1]
        m = jnp.dot(h2, wm_ref[...], preferred_element_type=jnp.float32) + bm_ref[...]
        t = jnp.dot(h2, ws_ref[...], preferred_element_type=jnp.float32) + bs_ref[...]
        sstd = _softplus(t)
        x_ref[...] = m + n_ref[...] * sstd
        blk_kl = jnp.sum(1.0 + 2.0 * jnp.log(sstd + 1e-8) - m * m - sstd * sstd)

        @pl.when(i == 0)
        def _():
            acc_ref[0] = 0.0

        acc_ref[0] += blk_kl

        @pl.when(i == nblk - 1)
        def _():
            kl_ref[...] = jnp.full((1, 1), -0.5 / N_NODES, jnp.float32) * acc_ref[0]

    return pl.pallas_call(
        body,
        grid=(nblk,),
        in_specs=[
            pl.BlockSpec((2, ROW_BLK, D), lambda i: (0, i, 0)),
            pl.BlockSpec((ROW_BLK, D), lambda i: (i, 0)),
            pl.BlockSpec((D, D), lambda i: (0, 0)),
            pl.BlockSpec((1, D), lambda i: (0, 0)),
            pl.BlockSpec((D, D), lambda i: (0, 0)),
            pl.BlockSpec((1, D), lambda i: (0, 0)),
        ],
        out_specs=[
            pl.BlockSpec((ROW_BLK, D), lambda i: (i, 0)),
            pl.BlockSpec((1, 1), lambda i: (0, 0)),
        ],
        out_shape=[
            jax.ShapeDtypeStruct((N_NODES, D), jnp.float32),
            jax.ShapeDtypeStruct((1, 1), jnp.float32),
        ],
        scratch_shapes=[pltpu.SMEM((1,), jnp.float32)],
    )(q, noise, W_mean, b_mean, W_std, b_std)


BPW = B // (NC * NS)


def _sc_decode(x, users, items, neg_items):
    @functools.partial(
        pl.kernel,
        out_type=(jax.ShapeDtypeStruct((B, D), jnp.float32),
                  jax.ShapeDtypeStruct((B, D), jnp.float32)),
        mesh=_mesh(),
        scratch_types=[
            pltpu.VMEM((BPW,), jnp.int32),
            pltpu.VMEM((BPW,), jnp.int32),
            pltpu.VMEM((BPW,), jnp.int32),
            pltpu.VMEM((BPW, D), jnp.float32),
            pltpu.VMEM((BPW, D), jnp.float32),
            pltpu.VMEM((BPW, D), jnp.float32),
            pltpu.SemaphoreType.DMA,
        ],
        compiler_params=_sc_compiler_params(),
    )
    def dec(x_h, u_h, it_h, ng_h, zp_h, zn_h, uv, iv, nv, xu, xi, xn, sem):
        c = lax.axis_index("c")
        s = lax.axis_index("s")
        base = (s * NC + c) * BPW
        pltpu.sync_copy(u_h.at[pl.ds(base, BPW)], uv)
        pltpu.sync_copy(it_h.at[pl.ds(base, BPW)], iv)
        pltpu.sync_copy(ng_h.at[pl.ds(base, BPW)], nv)
        off = jnp.full((LANES,), N_USER, jnp.int32)

        @pl.loop(0, BPW // LANES)
        def _(j):
            sl = pl.ds(j * LANES, LANES)
            iv[sl] = iv[sl] + off
            nv[sl] = nv[sl] + off

        pltpu.async_copy(x_h.at[uv], xu, sem).wait()
        pltpu.async_copy(x_h.at[iv], xi, sem).wait()
        pltpu.async_copy(x_h.at[nv], xn, sem).wait()

        @pl.loop(0, BPW)
        def _(r):
            for d in range(D // LANES):
                sl = pl.ds(d * LANES, LANES)
                u = xu[r, sl]
                xi[r, sl] = u * xi[r, sl]
                xn[r, sl] = u * xn[r, sl]

        pltpu.sync_copy(xi, zp_h.at[pl.ds(base, BPW)])
        pltpu.sync_copy(xn, zn_h.at[pl.ds(base, BPW)])

    return dec(x, users, items, neg_items)


def _tc_loss(zp, zn, Wd1, bd1, wd2, bd2, kl):
    def body(zp_ref, zn_ref, w1_ref, b1_ref, w2_ref, b2_ref, kl_ref, o_ref):
        w1 = w1_ref[...]
        b1 = b1_ref[...]
        w2 = w2_ref[...]
        hp = jnp.maximum(jnp.dot(zp_ref[...], w1, preferred_element_type=jnp.float32) + b1, 0.0)
        hn = jnp.maximum(jnp.dot(zn_ref[...], w1, preferred_element_type=jnp.float32) + b1, 0.0)
        lp = jnp.sum(hp * w2, axis=1, keepdims=True) + b2_ref[0, 0]
        ln = jnp.sum(hn * w2, axis=1, keepdims=True) + b2_ref[0, 0]
        lr = jnp.mean(_softplus(-lp)) + jnp.mean(_softplus(ln))
        o_ref[...] = jnp.full((1, 1), 1.0, jnp.float32) * (lr + 0.1 * kl_ref[0, 0])

    return pl.pallas_call(
        body,
        out_shape=jax.ShapeDtypeStruct((1, 1), jnp.float32),
    )(zp, zn, Wd1, bd1, wd2, bd2, kl)


def kernel(edge_values, emb, W_mean, b_mean, W_std, b_std, Wd1, bd1, Wd2, bd2,
           noise, edge_index, users, items, neg_items):
    row = edge_index[0]
    col = edge_index[1]
    p = _sc_spmm(row, col, edge_values, emb)
    h1 = _tc_add(p)
    q = _sc_spmm(row, col, edge_values, h1)
    x, kl = _tc_encode(q, noise, W_mean, b_mean.reshape(1, D),
                       W_std, b_std.reshape(1, D))
    zp, zn = _sc_decode(x, users, items, neg_items)
    loss = _tc_loss(zp, zn, Wd1, bd1.reshape(1, D), Wd2.reshape(1, D),
                    bd2.reshape(1, 1), kl)
    return loss[0, 0]
